# baseline (speedup 1.0000x reference)
import jax
import jax.numpy as jnp
from jax import lax
from jax.experimental import pallas as pl
from jax.experimental.pallas import tpu as pltpu
from jax.experimental.pallas import tpu_sc as plsc

MARGIN = 1.0
LANES = 16
BW = 128
CW = 64
NW = 32
PADDIM = 128


def _sc_distance_body(h_t, r_t, t_t, ent, rel, pos_out, negt_out, part_out,
                      ihv, irv, itv, distt_v, loss_v,
                      h0, r0, t0, h1, r1, t1,
                      sem_in, sem_g0, sem_g1):
    np1 = h_t.shape[0]
    dim = 64
    nc = lax.axis_size("c")
    wid = lax.axis_index("s") * nc + lax.axis_index("c")
    b0 = wid * BW

    row_ids = lax.iota(jnp.int32, LANES)

    cps = [pltpu.async_copy(src.at[:, pl.ds(b0, BW)], dst, sem_in)
           for src, dst in ((h_t, ihv), (r_t, irv), (t_t, itv))]
    for cp in cps:
        cp.wait()

    def fire(c, bufs, sem):
        n = c // 2
        j = (c % 2) * CW
        hB, rB, tB = bufs
        pltpu.async_copy(ent.at[ihv.at[n, pl.ds(j, CW)]], hB, sem)
        pltpu.async_copy(rel.at[irv.at[n, pl.ds(j, CW)]], rB, sem)
        pltpu.async_copy(ent.at[itv.at[n, pl.ds(j, CW)]], tB, sem)

    def drain(bufs, sem):
        hB, rB, tB = bufs
        pltpu.make_async_copy(ent.at[ihv.at[0, pl.ds(0, CW)]], hB, sem).wait()
        pltpu.make_async_copy(rel.at[irv.at[0, pl.ds(0, CW)]], rB, sem).wait()
        pltpu.make_async_copy(ent.at[itv.at[0, pl.ds(0, CW)]], tB, sem).wait()

    def compute(c, bufs):
        n = c // 2
        joff = (c % 2) * CW
        hB, rB, tB = bufs

        def gbody(g, carry):
            vec = jnp.zeros((LANES,), jnp.float32)
            for r in range(LANES):
                l = g * LANES + r
                acc = None
                for k in range(dim // LANES):
                    sl = pl.ds(k * LANES, LANES)
                    a = jnp.abs(hB[l, sl] + rB[l, sl] - tB[l, sl])
                    acc = a if acc is None else acc + a
                s = jnp.sum(acc)
                vec = jnp.where(row_ids == r, s, vec)
            distt_v[n, pl.ds(joff + g * LANES, LANES)] = vec
            return carry

        lax.fori_loop(0, CW // LANES, gbody, 0)

    bufs0 = (h0, r0, t0)
    bufs1 = (h1, r1, t1)
    n_chunks = np1 * 2

    fire(0, bufs0, sem_g0)

    def pair_body(p, carry):
        c0 = p * 2
        fire(c0 + 1, bufs1, sem_g1)
        drain(bufs0, sem_g0)
        compute(c0, bufs0)

        @pl.when(p + 1 < n_chunks // 2)
        def _():
            fire(c0 + 2, bufs0, sem_g0)

        drain(bufs1, sem_g1)
        compute(c0 + 1, bufs1)
        return carry

    lax.fori_loop(0, n_chunks // 2, pair_body, 0)

    def loss_body(n, acc):
        for g in range(BW // LANES):
            sl = pl.ds(g * LANES, LANES)
            acc = acc + jnp.maximum(distt_v[0, sl] - distt_v[n, sl] + MARGIN,
                                    0.0)
        return acc

    lacc = lax.fori_loop(1, np1, loss_body, jnp.zeros((LANES,), jnp.float32))
    loss_v[...] = lacc

    pltpu.sync_copy(distt_v.at[0], pos_out.at[pl.ds(b0, BW)])
    pltpu.sync_copy(distt_v.at[pl.ds(1, np1 - 1)],
                    negt_out.at[:, pl.ds(b0, BW)])
    pltpu.sync_copy(loss_v, part_out.at[wid])


def _loss_body(p_ref, denom_ref, loss_ref):
    loss_ref[0, 0] = jnp.sum(p_ref[...]) / denom_ref[0]


_TB = 256


def _transpose_pad_body(eye_ref, in_ref, out_ref):
    out_ref[:, 0:64] = lax.dot_general(
        eye_ref[...], in_ref[...], (((1,), (1,)), ((), ())),
        preferred_element_type=jnp.float32)


def _transpose_pad(table_t, eye):
    dim, n = table_t.shape
    grid = (n + _TB - 1) // _TB
    return pl.pallas_call(
        _transpose_pad_body,
        grid=(grid,),
        in_specs=[pl.BlockSpec((_TB, _TB), lambda j: (0, 0)),
                  pl.BlockSpec((dim, _TB), lambda j: (0, j))],
        out_specs=pl.BlockSpec((_TB, PADDIM), lambda j: (j, 0)),
        out_shape=jax.ShapeDtypeStruct((n, PADDIM), jnp.float32),
    )(eye, table_t)


@jax.jit
def kernel(triple_matrix, entities_emb, relations_emb):
    b, np1, _ = triple_matrix.shape
    neg_n = np1 - 1
    dim = entities_emb.shape[1]

    tri_t = jnp.transpose(triple_matrix, (2, 1, 0))
    h_t = tri_t[0]
    r_t = tri_t[1]
    t_t = tri_t[2]

    eye = jnp.eye(_TB, dtype=jnp.float32)
    ent128 = _transpose_pad(entities_emb.T, eye)
    rel128 = _transpose_pad(relations_emb.T, eye)

    mesh = plsc.VectorSubcoreMesh(core_axis_name="c", subcore_axis_name="s")
    pos, negt, partials = pl.kernel(
        _sc_distance_body,
        out_type=(
            jax.ShapeDtypeStruct((b,), jnp.float32),
            jax.ShapeDtypeStruct((neg_n, b), jnp.float32),
            jax.ShapeDtypeStruct((NW, LANES), jnp.float32),
        ),
        mesh=mesh,
        compiler_params=pltpu.CompilerParams(
            needs_layout_passes=False, use_tc_tiling_on_sc=True),
        scratch_types=[
            pltpu.VMEM((np1, BW), jnp.int32),
            pltpu.VMEM((np1, BW), jnp.int32),
            pltpu.VMEM((np1, BW), jnp.int32),
            pltpu.VMEM((np1, BW), jnp.float32),
            pltpu.VMEM((LANES,), jnp.float32),
            pltpu.VMEM((CW, PADDIM), jnp.float32),
            pltpu.VMEM((CW, PADDIM), jnp.float32),
            pltpu.VMEM((CW, PADDIM), jnp.float32),
            pltpu.VMEM((CW, PADDIM), jnp.float32),
            pltpu.VMEM((CW, PADDIM), jnp.float32),
            pltpu.VMEM((CW, PADDIM), jnp.float32),
            pltpu.SemaphoreType.DMA,
            pltpu.SemaphoreType.DMA,
            pltpu.SemaphoreType.DMA,
        ],
    )(h_t, r_t, t_t, ent128, rel128)

    denom = jnp.full((1,), float(b * neg_n), jnp.float32)
    loss = pl.pallas_call(
        _loss_body,
        out_shape=jax.ShapeDtypeStruct((1, 1), jnp.float32),
        in_specs=[pl.BlockSpec(memory_space=pltpu.VMEM),
                  pl.BlockSpec(memory_space=pltpu.SMEM)],
        out_specs=pl.BlockSpec(memory_space=pltpu.SMEM),
    )(partials, denom)[0, 0]

    return (loss, pos, negt.T)


# speedup vs baseline: 1.3504x; 1.3504x over previous
import jax
import jax.numpy as jnp
from jax import lax
from jax.experimental import pallas as pl
from jax.experimental.pallas import tpu as pltpu
from jax.experimental.pallas import tpu_sc as plsc

MARGIN = 1.0
LANES = 16
BW = 128
CW = 64
NW = 32
PADDIM = 128


def _sc_distance_body(h_t, r_t, t_t, ent, rel, pos_out, negt_out, part_out,
                      ihv, irv, itv, distt_v, loss_v,
                      h0, r0, t0, h1, r1, t1,
                      sem_in, sem_g0, sem_g1):
    np1 = h_t.shape[0]
    dim = 64
    nc = lax.axis_size("c")
    wid = lax.axis_index("s") * nc + lax.axis_index("c")
    b0 = wid * BW

    row_ids = lax.iota(jnp.int32, LANES)

    cps = [pltpu.async_copy(src.at[:, pl.ds(b0, BW)], dst, sem_in)
           for src, dst in ((h_t, ihv), (r_t, irv), (t_t, itv))]
    for cp in cps:
        cp.wait()

    def fire(c, bufs, sem):
        n = c // 2
        j = (c % 2) * CW
        hB, rB, tB = bufs
        pltpu.async_copy(ent.at[ihv.at[n, pl.ds(j, CW)]], hB, sem)
        pltpu.async_copy(rel.at[irv.at[n, pl.ds(j, CW)]], rB, sem)
        pltpu.async_copy(ent.at[itv.at[n, pl.ds(j, CW)]], tB, sem)

    def drain(bufs, sem):
        hB, rB, tB = bufs
        pltpu.make_async_copy(ent.at[ihv.at[0, pl.ds(0, CW)]], hB, sem).wait()
        pltpu.make_async_copy(rel.at[irv.at[0, pl.ds(0, CW)]], rB, sem).wait()
        pltpu.make_async_copy(ent.at[itv.at[0, pl.ds(0, CW)]], tB, sem).wait()

    def compute(c, bufs):
        n = c // 2
        joff = (c % 2) * CW
        hB, rB, tB = bufs

        def gbody(g, carry):
            vec = jnp.zeros((LANES,), jnp.float32)
            for r in range(LANES):
                l = g * LANES + r
                acc = None
                for k in range(dim // LANES):
                    sl = pl.ds(k * LANES, LANES)
                    a = jnp.abs(hB[l, sl] + rB[l, sl] - tB[l, sl])
                    acc = a if acc is None else acc + a
                s = jnp.sum(acc)
                vec = jnp.where(row_ids == r, s, vec)
            distt_v[n, pl.ds(joff + g * LANES, LANES)] = vec
            return carry

        lax.fori_loop(0, CW // LANES, gbody, 0)

    bufs0 = (h0, r0, t0)
    bufs1 = (h1, r1, t1)
    n_chunks = np1 * 2

    fire(0, bufs0, sem_g0)

    def pair_body(p, carry):
        c0 = p * 2
        fire(c0 + 1, bufs1, sem_g1)
        drain(bufs0, sem_g0)
        compute(c0, bufs0)

        @pl.when(p + 1 < n_chunks // 2)
        def _():
            fire(c0 + 2, bufs0, sem_g0)

        drain(bufs1, sem_g1)
        compute(c0 + 1, bufs1)
        return carry

    lax.fori_loop(0, n_chunks // 2, pair_body, 0)

    def loss_body(n, acc):
        for g in range(BW // LANES):
            sl = pl.ds(g * LANES, LANES)
            acc = acc + jnp.maximum(distt_v[0, sl] - distt_v[n, sl] + MARGIN,
                                    0.0)
        return acc

    lacc = lax.fori_loop(1, np1, loss_body, jnp.zeros((LANES,), jnp.float32))
    loss_v[...] = lacc

    pltpu.sync_copy(distt_v.at[0], pos_out.at[pl.ds(b0, BW)])
    pltpu.sync_copy(distt_v.at[pl.ds(1, np1 - 1)],
                    negt_out.at[:, pl.ds(b0, BW)])
    pltpu.sync_copy(loss_v, part_out.at[wid])


TW = 256


def _sc_transpose_body(ent_t, rel_t, tail_e, tail_r, entp, relp,
                       slab0, dst0, slab1, dst1, sem0, sem1):
    n = ent_t.shape[1]
    nfull = n // TW
    nc = lax.axis_size("c")
    wid = lax.axis_index("s") * nc + lax.axis_index("c")
    iot = lax.iota(jnp.int32, LANES)
    dvecs = [iot + LANES * k for k in range(4)]

    def do_table(src, dsth):
        def fire(t, slab, sem):
            win = wid + NW * t

            @pl.when(win < nfull)
            def _():
                pltpu.async_copy(src.at[:, pl.ds(win * TW, TW)], slab, sem)

        def wait_slab(t, slab, sem):
            @pl.when(wid + NW * t < nfull)
            def _():
                pltpu.make_async_copy(src.at[:, pl.ds(0, TW)], slab,
                                      sem).wait()

        def process(t, slab, dstv):
            win = wid + NW * t

            @pl.when(win < nfull)
            def _():
                def ebody(e4, carry):
                    for u in range(4):
                        el = e4 * 4 + u
                        ev = jnp.full((LANES,), el, jnp.int32)
                        for k in range(4):
                            v = plsc.load_gather(slab, [dvecs[k], ev])
                            dstv[el, pl.ds(k * LANES, LANES)] = v
                    return carry

                lax.fori_loop(0, TW // 4, ebody, 0)
                pltpu.sync_copy(dstv, dsth.at[pl.ds(win * TW, TW)])

        fire(0, slab0, sem0)

        def pbody(p, carry):
            t0 = 2 * p
            fire(t0 + 1, slab1, sem1)
            wait_slab(t0, slab0, sem0)
            process(t0, slab0, dst0)
            fire(t0 + 2, slab0, sem0)
            wait_slab(t0 + 1, slab1, sem1)
            process(t0 + 1, slab1, dst1)
            return carry

        n_t = (nfull + NW - 1) // NW + 1
        lax.fori_loop(0, (n_t + 1) // 2, pbody, 0)

    do_table(ent_t, entp)
    do_table(rel_t, relp)

    tail = n - nfull * TW

    @pl.when(wid == 0)
    def _():
        pltpu.sync_copy(tail_e, dst0.at[pl.ds(0, tail)])
        pltpu.sync_copy(dst0.at[pl.ds(0, tail)],
                        entp.at[pl.ds(nfull * TW, tail)])

    @pl.when(wid == 1)
    def _():
        pltpu.sync_copy(tail_r, dst0.at[pl.ds(0, tail)])
        pltpu.sync_copy(dst0.at[pl.ds(0, tail)],
                        relp.at[pl.ds(nfull * TW, tail)])


def _sc_transpose(ent_t, rel_t, tail_e, tail_r):
    n = ent_t.shape[1]
    mesh = plsc.VectorSubcoreMesh(core_axis_name="c", subcore_axis_name="s")
    return pl.kernel(
        _sc_transpose_body,
        out_type=(
            jax.ShapeDtypeStruct((n, PADDIM), jnp.float32),
            jax.ShapeDtypeStruct((n, PADDIM), jnp.float32),
        ),
        mesh=mesh,
        compiler_params=pltpu.CompilerParams(
            needs_layout_passes=False, use_tc_tiling_on_sc=True),
        scratch_types=[
            pltpu.VMEM((64, TW), jnp.float32),
            pltpu.VMEM((TW, PADDIM), jnp.float32),
            pltpu.VMEM((64, TW), jnp.float32),
            pltpu.VMEM((TW, PADDIM), jnp.float32),
            pltpu.SemaphoreType.DMA,
            pltpu.SemaphoreType.DMA,
        ],
    )(ent_t, rel_t, tail_e, tail_r)


def _loss_body(p_ref, denom_ref, loss_ref):
    loss_ref[0, 0] = jnp.sum(p_ref[...]) / denom_ref[0]


@jax.jit
def kernel(triple_matrix, entities_emb, relations_emb):
    b, np1, _ = triple_matrix.shape
    neg_n = np1 - 1
    dim = entities_emb.shape[1]

    tri_t = jnp.transpose(triple_matrix, (2, 1, 0))
    h_t = tri_t[0]
    r_t = tri_t[1]
    t_t = tri_t[2]

    n_ent = entities_emb.shape[0]
    ncut = (n_ent // TW) * TW
    tail_e = jnp.pad(entities_emb[ncut:], ((0, 0), (0, PADDIM - dim)))
    tail_r = jnp.pad(relations_emb[ncut:], ((0, 0), (0, PADDIM - dim)))
    ent128, rel128 = _sc_transpose(entities_emb.T, relations_emb.T,
                                   tail_e, tail_r)

    mesh = plsc.VectorSubcoreMesh(core_axis_name="c", subcore_axis_name="s")
    pos, negt, partials = pl.kernel(
        _sc_distance_body,
        out_type=(
            jax.ShapeDtypeStruct((b,), jnp.float32),
            jax.ShapeDtypeStruct((neg_n, b), jnp.float32),
            jax.ShapeDtypeStruct((NW, LANES), jnp.float32),
        ),
        mesh=mesh,
        compiler_params=pltpu.CompilerParams(
            needs_layout_passes=False, use_tc_tiling_on_sc=True),
        scratch_types=[
            pltpu.VMEM((np1, BW), jnp.int32),
            pltpu.VMEM((np1, BW), jnp.int32),
            pltpu.VMEM((np1, BW), jnp.int32),
            pltpu.VMEM((np1, BW), jnp.float32),
            pltpu.VMEM((LANES,), jnp.float32),
            pltpu.VMEM((CW, PADDIM), jnp.float32),
            pltpu.VMEM((CW, PADDIM), jnp.float32),
            pltpu.VMEM((CW, PADDIM), jnp.float32),
            pltpu.VMEM((CW, PADDIM), jnp.float32),
            pltpu.VMEM((CW, PADDIM), jnp.float32),
            pltpu.VMEM((CW, PADDIM), jnp.float32),
            pltpu.SemaphoreType.DMA,
            pltpu.SemaphoreType.DMA,
            pltpu.SemaphoreType.DMA,
        ],
    )(h_t, r_t, t_t, ent128, rel128)

    denom = jnp.full((1,), float(b * neg_n), jnp.float32)
    loss = pl.pallas_call(
        _loss_body,
        out_shape=jax.ShapeDtypeStruct((1, 1), jnp.float32),
        in_specs=[pl.BlockSpec(memory_space=pltpu.VMEM),
                  pl.BlockSpec(memory_space=pltpu.SMEM)],
        out_specs=pl.BlockSpec(memory_space=pltpu.SMEM),
    )(partials, denom)[0, 0]

    return (loss, pos, negt.T)


# speedup vs baseline: 1.6939x; 1.2544x over previous
import jax
import jax.numpy as jnp
from jax import lax
from jax.experimental import pallas as pl
from jax.experimental.pallas import tpu as pltpu
from jax.experimental.pallas import tpu_sc as plsc

MARGIN = 1.0
LANES = 16
BW = 128
CW = 64
NW = 32
PADDIM = 128


def _sc_distance_body(h_t, r_t, t_t, ent, rel, pos_out, negt_out, part_out,
                      ihv, irv, itv, hhv, hrv, htv, distt_v, loss_v,
                      h0, r0, t0, h1, r1, t1,
                      sem_in, sem_g0, sem_g1):
    np1 = h_t.shape[0]
    dim = 64
    nc = lax.axis_size("c")
    wid = lax.axis_index("s") * nc + lax.axis_index("c")
    b0 = wid * BW

    row_ids = lax.iota(jnp.int32, LANES)

    cps = [pltpu.async_copy(src.at[:, pl.ds(b0, BW)], dst, sem_in)
           for src, dst in ((h_t, ihv), (r_t, irv), (t_t, itv))]
    for cp in cps:
        cp.wait()

    def split_body(nn, carry):
        for g in range(BW // LANES):
            sl = pl.ds(g * LANES, LANES)
            for iv, hv in ((ihv, hhv), (irv, hrv), (itv, htv)):
                v = iv[nn, sl]
                hv[nn, sl] = (v & 1) * dim
                iv[nn, sl] = v >> 1
        return carry

    lax.fori_loop(0, np1, split_body, 0)

    def fire(c, bufs, sem):
        n = c // 2
        j = (c % 2) * CW
        hB, rB, tB = bufs
        pltpu.async_copy(ent.at[ihv.at[n, pl.ds(j, CW)]], hB, sem)
        pltpu.async_copy(rel.at[irv.at[n, pl.ds(j, CW)]], rB, sem)
        pltpu.async_copy(ent.at[itv.at[n, pl.ds(j, CW)]], tB, sem)

    def drain(bufs, sem):
        hB, rB, tB = bufs
        pltpu.make_async_copy(ent.at[ihv.at[0, pl.ds(0, CW)]], hB, sem).wait()
        pltpu.make_async_copy(rel.at[irv.at[0, pl.ds(0, CW)]], rB, sem).wait()
        pltpu.make_async_copy(ent.at[itv.at[0, pl.ds(0, CW)]], tB, sem).wait()

    def compute(c, bufs):
        n = c // 2
        joff = (c % 2) * CW
        hB, rB, tB = bufs

        def gbody(g, carry):
            gsl = pl.ds(joff + g * LANES, LANES)
            hh16 = hhv[n, gsl]
            hr16 = hrv[n, gsl]
            ht16 = htv[n, gsl]
            vec = jnp.zeros((LANES,), jnp.float32)
            for r in range(LANES):
                l = g * LANES + r
                ho = hh16[r]
                ro = hr16[r]
                to = ht16[r]
                acc = None
                for k in range(dim // LANES):
                    o = k * LANES
                    a = jnp.abs(hB[l, pl.ds(ho + o, LANES)]
                                + rB[l, pl.ds(ro + o, LANES)]
                                - tB[l, pl.ds(to + o, LANES)])
                    acc = a if acc is None else acc + a
                s = jnp.sum(acc)
                vec = jnp.where(row_ids == r, s, vec)
            distt_v[n, gsl] = vec
            return carry

        lax.fori_loop(0, CW // LANES, gbody, 0)

    bufs0 = (h0, r0, t0)
    bufs1 = (h1, r1, t1)
    n_chunks = np1 * 2

    fire(0, bufs0, sem_g0)

    def pair_body(p, carry):
        c0 = p * 2
        fire(c0 + 1, bufs1, sem_g1)
        drain(bufs0, sem_g0)
        compute(c0, bufs0)

        @pl.when(p + 1 < n_chunks // 2)
        def _():
            fire(c0 + 2, bufs0, sem_g0)

        drain(bufs1, sem_g1)
        compute(c0 + 1, bufs1)
        return carry

    lax.fori_loop(0, n_chunks // 2, pair_body, 0)

    def loss_body(n, acc):
        for g in range(BW // LANES):
            sl = pl.ds(g * LANES, LANES)
            acc = acc + jnp.maximum(distt_v[0, sl] - distt_v[n, sl] + MARGIN,
                                    0.0)
        return acc

    lacc = lax.fori_loop(1, np1, loss_body, jnp.zeros((LANES,), jnp.float32))
    loss_v[...] = lacc

    pltpu.sync_copy(distt_v.at[0], pos_out.at[pl.ds(b0, BW)])
    pltpu.sync_copy(distt_v.at[pl.ds(1, np1 - 1)],
                    negt_out.at[:, pl.ds(b0, BW)])
    pltpu.sync_copy(loss_v, part_out.at[wid])


TW = 256


def _sc_transpose_body(ent_t, rel_t, tail_e, tail_r, entp, relp,
                       slab0, dst0, slab1, dst1, sem0, sem1):
    n = ent_t.shape[1]
    nfull = n // TW
    nc = lax.axis_size("c")
    wid = lax.axis_index("s") * nc + lax.axis_index("c")
    iot = lax.iota(jnp.int32, LANES)
    pvec64 = (iot & 1) * 64
    rvecs = [(iot + LANES * j) // 2 for j in range(TW // LANES)]

    def do_table(src, dsth):
        def fire(t, slab, sem):
            win = wid + NW * t

            @pl.when(win < nfull)
            def _():
                pltpu.async_copy(src.at[:, pl.ds(win * TW, TW)], slab, sem)

        def wait_slab(t, slab, sem):
            @pl.when(wid + NW * t < nfull)
            def _():
                pltpu.make_async_copy(src.at[:, pl.ds(0, TW)], slab,
                                      sem).wait()

        def process(t, slab, dstv):
            win = wid + NW * t

            @pl.when(win < nfull)
            def _():
                def dbody(d2, carry):
                    for u in range(2):
                        d = d2 * 2 + u
                        cv = pvec64 + d
                        for j in range(TW // LANES):
                            v = slab[d, pl.ds(j * LANES, LANES)]
                            plsc.store_scatter(dstv, [rvecs[j], cv], v)
                    return carry

                lax.fori_loop(0, 32, dbody, 0)
                pltpu.sync_copy(dstv, dsth.at[pl.ds(win * (TW // 2),
                                                    TW // 2)])

        fire(0, slab0, sem0)

        def pbody(p, carry):
            t0 = 2 * p
            fire(t0 + 1, slab1, sem1)
            wait_slab(t0, slab0, sem0)
            process(t0, slab0, dst0)
            fire(t0 + 2, slab0, sem0)
            wait_slab(t0 + 1, slab1, sem1)
            process(t0 + 1, slab1, dst1)
            return carry

        n_t = (nfull + NW - 1) // NW + 1
        lax.fori_loop(0, (n_t + 1) // 2, pbody, 0)

    do_table(ent_t, entp)
    do_table(rel_t, relp)

    tailp = tail_e.shape[0]
    prow0 = nfull * (TW // 2)

    @pl.when(wid == 0)
    def _():
        pltpu.sync_copy(tail_e, dst0.at[pl.ds(0, tailp)])
        pltpu.sync_copy(dst0.at[pl.ds(0, tailp)],
                        entp.at[pl.ds(prow0, tailp)])

    @pl.when(wid == 1)
    def _():
        pltpu.sync_copy(tail_r, dst0.at[pl.ds(0, tailp)])
        pltpu.sync_copy(dst0.at[pl.ds(0, tailp)],
                        relp.at[pl.ds(prow0, tailp)])


def _sc_transpose(ent_t, rel_t, tail_e, tail_r):
    n = ent_t.shape[1]
    n2 = (n + 1) // 2
    mesh = plsc.VectorSubcoreMesh(core_axis_name="c", subcore_axis_name="s")
    return pl.kernel(
        _sc_transpose_body,
        out_type=(
            jax.ShapeDtypeStruct((n2, PADDIM), jnp.float32),
            jax.ShapeDtypeStruct((n2, PADDIM), jnp.float32),
        ),
        mesh=mesh,
        compiler_params=pltpu.CompilerParams(
            needs_layout_passes=False, use_tc_tiling_on_sc=True),
        scratch_types=[
            pltpu.VMEM((64, TW), jnp.float32),
            pltpu.VMEM((TW // 2, PADDIM), jnp.float32),
            pltpu.VMEM((64, TW), jnp.float32),
            pltpu.VMEM((TW // 2, PADDIM), jnp.float32),
            pltpu.SemaphoreType.DMA,
            pltpu.SemaphoreType.DMA,
        ],
    )(ent_t, rel_t, tail_e, tail_r)


def _loss_body(p_ref, denom_ref, loss_ref):
    loss_ref[0, 0] = jnp.sum(p_ref[...]) / denom_ref[0]


@jax.jit
def kernel(triple_matrix, entities_emb, relations_emb):
    b, np1, _ = triple_matrix.shape
    neg_n = np1 - 1
    dim = entities_emb.shape[1]

    tri_t = jnp.transpose(triple_matrix, (2, 1, 0))
    h_t = tri_t[0]
    r_t = tri_t[1]
    t_t = tri_t[2]

    n_ent = entities_emb.shape[0]
    ncut = (n_ent // TW) * TW
    tail_n = n_ent - ncut

    def pack_tail(t):
        t = jnp.pad(t, ((0, tail_n % 2), (0, 0)))
        return t.reshape(-1, 2 * dim)

    tail_e = pack_tail(entities_emb[ncut:])
    tail_r = pack_tail(relations_emb[ncut:])
    ent128, rel128 = _sc_transpose(entities_emb.T, relations_emb.T,
                                   tail_e, tail_r)

    mesh = plsc.VectorSubcoreMesh(core_axis_name="c", subcore_axis_name="s")
    pos, negt, partials = pl.kernel(
        _sc_distance_body,
        out_type=(
            jax.ShapeDtypeStruct((b,), jnp.float32),
            jax.ShapeDtypeStruct((neg_n, b), jnp.float32),
            jax.ShapeDtypeStruct((NW, LANES), jnp.float32),
        ),
        mesh=mesh,
        compiler_params=pltpu.CompilerParams(
            needs_layout_passes=False, use_tc_tiling_on_sc=True),
        scratch_types=[
            pltpu.VMEM((np1, BW), jnp.int32),
            pltpu.VMEM((np1, BW), jnp.int32),
            pltpu.VMEM((np1, BW), jnp.int32),
            pltpu.VMEM((np1, BW), jnp.int32),
            pltpu.VMEM((np1, BW), jnp.int32),
            pltpu.VMEM((np1, BW), jnp.int32),
            pltpu.VMEM((np1, BW), jnp.float32),
            pltpu.VMEM((LANES,), jnp.float32),
            pltpu.VMEM((CW, PADDIM), jnp.float32),
            pltpu.VMEM((CW, PADDIM), jnp.float32),
            pltpu.VMEM((CW, PADDIM), jnp.float32),
            pltpu.VMEM((CW, PADDIM), jnp.float32),
            pltpu.VMEM((CW, PADDIM), jnp.float32),
            pltpu.VMEM((CW, PADDIM), jnp.float32),
            pltpu.SemaphoreType.DMA,
            pltpu.SemaphoreType.DMA,
            pltpu.SemaphoreType.DMA,
        ],
    )(h_t, r_t, t_t, ent128, rel128)

    denom = jnp.full((1,), float(b * neg_n), jnp.float32)
    loss = pl.pallas_call(
        _loss_body,
        out_shape=jax.ShapeDtypeStruct((1, 1), jnp.float32),
        in_specs=[pl.BlockSpec(memory_space=pltpu.VMEM),
                  pl.BlockSpec(memory_space=pltpu.SMEM)],
        out_specs=pl.BlockSpec(memory_space=pltpu.SMEM),
    )(partials, denom)[0, 0]

    return (loss, pos, negt.T)


# speedup vs baseline: 2.2563x; 1.3320x over previous
import jax
import jax.numpy as jnp
from jax import lax
from jax.experimental import pallas as pl
from jax.experimental.pallas import tpu as pltpu
from jax.experimental.pallas import tpu_sc as plsc

MARGIN = 1.0
LANES = 16
BW = 128
CW = 64
NW = 32
PADDIM = 128


def _sc_distance_body(h_t, r_t, t_t, ent, rel, pos_out, negt_out, part_out,
                      ihv, irv, itv, hhv, hrv, htv, distt_v, loss_v,
                      h0, r0, t0, h1, r1, t1,
                      sem_in, sem_g0, sem_g1):
    np1 = h_t.shape[0]
    dim = 64
    nc = lax.axis_size("c")
    wid = lax.axis_index("s") * nc + lax.axis_index("c")
    b0 = wid * BW

    row_ids = lax.iota(jnp.int32, LANES)

    cps = [pltpu.async_copy(src.at[:, pl.ds(b0, BW)], dst, sem_in)
           for src, dst in ((h_t, ihv), (r_t, irv), (t_t, itv))]
    for cp in cps:
        cp.wait()

    def split_body(nn, carry):
        for g in range(BW // LANES):
            sl = pl.ds(g * LANES, LANES)
            for iv, hv in ((ihv, hhv), (irv, hrv), (itv, htv)):
                v = iv[nn, sl]
                hv[nn, sl] = (v & 1) * dim
                iv[nn, sl] = v >> 1
        return carry

    lax.fori_loop(0, np1, split_body, 0)

    def fire(c, bufs, sem):
        n = c // 2
        j = (c % 2) * CW
        hB, rB, tB = bufs
        pltpu.async_copy(ent.at[ihv.at[n, pl.ds(j, CW)]], hB, sem)
        pltpu.async_copy(rel.at[irv.at[n, pl.ds(j, CW)]], rB, sem)
        pltpu.async_copy(ent.at[itv.at[n, pl.ds(j, CW)]], tB, sem)

    def drain(bufs, sem):
        hB, rB, tB = bufs
        pltpu.make_async_copy(ent.at[ihv.at[0, pl.ds(0, CW)]], hB, sem).wait()
        pltpu.make_async_copy(rel.at[irv.at[0, pl.ds(0, CW)]], rB, sem).wait()
        pltpu.make_async_copy(ent.at[itv.at[0, pl.ds(0, CW)]], tB, sem).wait()

    def compute(c, bufs):
        n = c // 2
        joff = (c % 2) * CW
        hB, rB, tB = bufs

        def gbody(g, carry):
            gsl = pl.ds(joff + g * LANES, LANES)
            hh16 = hhv[n, gsl]
            hr16 = hrv[n, gsl]
            ht16 = htv[n, gsl]
            vec = jnp.zeros((LANES,), jnp.float32)
            for r in range(LANES):
                l = g * LANES + r
                ho = hh16[r]
                ro = hr16[r]
                to = ht16[r]
                acc = None
                for k in range(dim // LANES):
                    o = k * LANES
                    a = jnp.abs(hB[l, pl.ds(ho + o, LANES)]
                                + rB[l, pl.ds(ro + o, LANES)]
                                - tB[l, pl.ds(to + o, LANES)])
                    acc = a if acc is None else acc + a
                s = jnp.sum(acc)
                vec = jnp.where(row_ids == r, s, vec)
            distt_v[n, gsl] = vec
            return carry

        lax.fori_loop(0, CW // LANES, gbody, 0)

    bufs0 = (h0, r0, t0)
    bufs1 = (h1, r1, t1)
    n_chunks = np1 * 2

    fire(0, bufs0, sem_g0)

    def pair_body(p, carry):
        c0 = p * 2
        fire(c0 + 1, bufs1, sem_g1)
        drain(bufs0, sem_g0)
        compute(c0, bufs0)

        @pl.when(p + 1 < n_chunks // 2)
        def _():
            fire(c0 + 2, bufs0, sem_g0)

        drain(bufs1, sem_g1)
        compute(c0 + 1, bufs1)
        return carry

    lax.fori_loop(0, n_chunks // 2, pair_body, 0)

    def loss_body(n, acc):
        for g in range(BW // LANES):
            sl = pl.ds(g * LANES, LANES)
            acc = acc + jnp.maximum(distt_v[0, sl] - distt_v[n, sl] + MARGIN,
                                    0.0)
        return acc

    lacc = lax.fori_loop(1, np1, loss_body, jnp.zeros((LANES,), jnp.float32))
    loss_v[...] = lacc

    pltpu.sync_copy(distt_v.at[0], pos_out.at[pl.ds(b0, BW)])
    pltpu.sync_copy(distt_v.at[pl.ds(1, np1 - 1)],
                    negt_out.at[:, pl.ds(b0, BW)])
    pltpu.sync_copy(loss_v, part_out.at[wid])


TW = 256


def _sc_transpose_body(ent_t, rel_t, tail_e, tail_r, entp, relp,
                       slab0, dst0, slab1, dst1, sem0, sem1):
    n = ent_t.shape[1]
    nfull = n // TW
    nc = lax.axis_size("c")
    wid = lax.axis_index("s") * nc + lax.axis_index("c")
    iot = lax.iota(jnp.int32, LANES)
    pvec64 = (iot & 1) * 64
    rvecs = [(iot + LANES * j) // 2 for j in range(TW // LANES)]

    def do_table(src, dsth):
        def fire(t, slab, sem):
            win = wid + NW * t

            @pl.when(win < nfull)
            def _():
                pltpu.async_copy(src.at[:, pl.ds(win * TW, TW)], slab, sem)

        def wait_slab(t, slab, sem):
            @pl.when(wid + NW * t < nfull)
            def _():
                pltpu.make_async_copy(src.at[:, pl.ds(0, TW)], slab,
                                      sem).wait()

        def process(t, slab, dstv):
            win = wid + NW * t

            @pl.when(win < nfull)
            def _():
                @plsc.parallel_loop(0, 64, unroll=2)
                def _(d):
                    cv = pvec64 + d
                    for j in range(TW // LANES):
                        v = slab[d, pl.ds(j * LANES, LANES)]
                        plsc.store_scatter(dstv, [rvecs[j], cv], v)
                pltpu.sync_copy(dstv, dsth.at[pl.ds(win * (TW // 2),
                                                    TW // 2)])

        fire(0, slab0, sem0)

        def pbody(p, carry):
            t0 = 2 * p
            fire(t0 + 1, slab1, sem1)
            wait_slab(t0, slab0, sem0)
            process(t0, slab0, dst0)
            fire(t0 + 2, slab0, sem0)
            wait_slab(t0 + 1, slab1, sem1)
            process(t0 + 1, slab1, dst1)
            return carry

        n_t = (nfull + NW - 1) // NW + 1
        lax.fori_loop(0, (n_t + 1) // 2, pbody, 0)

    do_table(ent_t, entp)
    do_table(rel_t, relp)

    tailp = tail_e.shape[0]
    prow0 = nfull * (TW // 2)

    @pl.when(wid == 0)
    def _():
        pltpu.sync_copy(tail_e, dst0.at[pl.ds(0, tailp)])
        pltpu.sync_copy(dst0.at[pl.ds(0, tailp)],
                        entp.at[pl.ds(prow0, tailp)])

    @pl.when(wid == 1)
    def _():
        pltpu.sync_copy(tail_r, dst0.at[pl.ds(0, tailp)])
        pltpu.sync_copy(dst0.at[pl.ds(0, tailp)],
                        relp.at[pl.ds(prow0, tailp)])


def _sc_transpose(ent_t, rel_t, tail_e, tail_r):
    n = ent_t.shape[1]
    n2 = (n + 1) // 2
    mesh = plsc.VectorSubcoreMesh(core_axis_name="c", subcore_axis_name="s")
    return pl.kernel(
        _sc_transpose_body,
        out_type=(
            jax.ShapeDtypeStruct((n2, PADDIM), jnp.float32),
            jax.ShapeDtypeStruct((n2, PADDIM), jnp.float32),
        ),
        mesh=mesh,
        compiler_params=pltpu.CompilerParams(
            needs_layout_passes=False, use_tc_tiling_on_sc=True),
        scratch_types=[
            pltpu.VMEM((64, TW), jnp.float32),
            pltpu.VMEM((TW // 2, PADDIM), jnp.float32),
            pltpu.VMEM((64, TW), jnp.float32),
            pltpu.VMEM((TW // 2, PADDIM), jnp.float32),
            pltpu.SemaphoreType.DMA,
            pltpu.SemaphoreType.DMA,
        ],
    )(ent_t, rel_t, tail_e, tail_r)


def _loss_body(p_ref, denom_ref, loss_ref):
    loss_ref[0, 0] = jnp.sum(p_ref[...]) / denom_ref[0]


@jax.jit
def kernel(triple_matrix, entities_emb, relations_emb):
    b, np1, _ = triple_matrix.shape
    neg_n = np1 - 1
    dim = entities_emb.shape[1]

    tri_t = jnp.transpose(triple_matrix, (2, 1, 0))
    h_t = tri_t[0]
    r_t = tri_t[1]
    t_t = tri_t[2]

    n_ent = entities_emb.shape[0]
    ncut = (n_ent // TW) * TW
    tail_n = n_ent - ncut

    def pack_tail(t):
        t = jnp.pad(t, ((0, tail_n % 2), (0, 0)))
        return t.reshape(-1, 2 * dim)

    tail_e = pack_tail(entities_emb[ncut:])
    tail_r = pack_tail(relations_emb[ncut:])
    ent128, rel128 = _sc_transpose(entities_emb.T, relations_emb.T,
                                   tail_e, tail_r)

    mesh = plsc.VectorSubcoreMesh(core_axis_name="c", subcore_axis_name="s")
    pos, negt, partials = pl.kernel(
        _sc_distance_body,
        out_type=(
            jax.ShapeDtypeStruct((b,), jnp.float32),
            jax.ShapeDtypeStruct((neg_n, b), jnp.float32),
            jax.ShapeDtypeStruct((NW, LANES), jnp.float32),
        ),
        mesh=mesh,
        compiler_params=pltpu.CompilerParams(
            needs_layout_passes=False, use_tc_tiling_on_sc=True),
        scratch_types=[
            pltpu.VMEM((np1, BW), jnp.int32),
            pltpu.VMEM((np1, BW), jnp.int32),
            pltpu.VMEM((np1, BW), jnp.int32),
            pltpu.VMEM((np1, BW), jnp.int32),
            pltpu.VMEM((np1, BW), jnp.int32),
            pltpu.VMEM((np1, BW), jnp.int32),
            pltpu.VMEM((np1, BW), jnp.float32),
            pltpu.VMEM((LANES,), jnp.float32),
            pltpu.VMEM((CW, PADDIM), jnp.float32),
            pltpu.VMEM((CW, PADDIM), jnp.float32),
            pltpu.VMEM((CW, PADDIM), jnp.float32),
            pltpu.VMEM((CW, PADDIM), jnp.float32),
            pltpu.VMEM((CW, PADDIM), jnp.float32),
            pltpu.VMEM((CW, PADDIM), jnp.float32),
            pltpu.SemaphoreType.DMA,
            pltpu.SemaphoreType.DMA,
            pltpu.SemaphoreType.DMA,
        ],
    )(h_t, r_t, t_t, ent128, rel128)

    denom = jnp.full((1,), float(b * neg_n), jnp.float32)
    loss = pl.pallas_call(
        _loss_body,
        out_shape=jax.ShapeDtypeStruct((1, 1), jnp.float32),
        in_specs=[pl.BlockSpec(memory_space=pltpu.VMEM),
                  pl.BlockSpec(memory_space=pltpu.SMEM)],
        out_specs=pl.BlockSpec(memory_space=pltpu.SMEM),
    )(partials, denom)[0, 0]

    return (loss, pos, negt.T)


# speedup vs baseline: 3.8067x; 1.6871x over previous
import jax
import jax.numpy as jnp
from jax import lax
from jax.experimental import pallas as pl
from jax.experimental.pallas import tpu as pltpu
from jax.experimental.pallas import tpu_sc as plsc

MARGIN = 1.0
LANES = 16
BW = 128
CW = 64
NW = 32
PADDIM = 128


def _sc_distance_body(h_t, r_t, t_t, ent, rel, pos_out, negt_out, part_out,
                      ihv, irv, itv, distt_v, loss_v,
                      h0, r0, t0, h1, r1, t1,
                      sem_in, sem_g0, sem_g1):
    np1 = h_t.shape[0]
    dim = 64
    nc = lax.axis_size("c")
    wid = lax.axis_index("s") * nc + lax.axis_index("c")
    b0 = wid * BW

    row_ids = lax.iota(jnp.int32, LANES)

    cps = [pltpu.async_copy(src.at[:, pl.ds(b0, BW)], dst, sem_in)
           for src, dst in ((h_t, ihv), (r_t, irv), (t_t, itv))]
    for cp in cps:
        cp.wait()

    def fire(c, bufs, sem):
        n = c // 2
        j = (c % 2) * CW
        hB, rB, tB = bufs
        pltpu.async_copy(ent.at[ihv.at[n, pl.ds(j, CW)]], hB, sem)
        pltpu.async_copy(rel.at[irv.at[n, pl.ds(j, CW)]], rB, sem)
        pltpu.async_copy(ent.at[itv.at[n, pl.ds(j, CW)]], tB, sem)

    def drain(bufs, sem):
        hB, rB, tB = bufs
        pltpu.make_async_copy(ent.at[ihv.at[0, pl.ds(0, CW)]], hB, sem).wait()
        pltpu.make_async_copy(rel.at[irv.at[0, pl.ds(0, CW)]], rB, sem).wait()
        pltpu.make_async_copy(ent.at[itv.at[0, pl.ds(0, CW)]], tB, sem).wait()

    def compute(c, bufs):
        n = c // 2
        joff = (c % 2) * CW
        hB, rB, tB = bufs

        def gbody(g, carry):
            vec = jnp.zeros((LANES,), jnp.float32)
            for r in range(LANES):
                l = g * LANES + r
                acc = None
                for k in range(dim // LANES):
                    sl = pl.ds(k * LANES, LANES)
                    a = jnp.abs(hB[l, sl] + rB[l, sl] - tB[l, sl])
                    acc = a if acc is None else acc + a
                s = jnp.sum(acc)
                vec = jnp.where(row_ids == r, s, vec)
            distt_v[n, pl.ds(joff + g * LANES, LANES)] = vec
            return carry

        lax.fori_loop(0, CW // LANES, gbody, 0)

    bufs0 = (h0, r0, t0)
    bufs1 = (h1, r1, t1)
    n_chunks = np1 * 2

    fire(0, bufs0, sem_g0)

    def pair_body(p, carry):
        c0 = p * 2
        fire(c0 + 1, bufs1, sem_g1)
        drain(bufs0, sem_g0)
        compute(c0, bufs0)

        @pl.when(p + 1 < n_chunks // 2)
        def _():
            fire(c0 + 2, bufs0, sem_g0)

        drain(bufs1, sem_g1)
        compute(c0 + 1, bufs1)
        return carry

    lax.fori_loop(0, n_chunks // 2, pair_body, 0)

    def loss_body(n, acc):
        for g in range(BW // LANES):
            sl = pl.ds(g * LANES, LANES)
            acc = acc + jnp.maximum(distt_v[0, sl] - distt_v[n, sl] + MARGIN,
                                    0.0)
        return acc

    lacc = lax.fori_loop(1, np1, loss_body, jnp.zeros((LANES,), jnp.float32))
    loss_v[...] = lacc

    pltpu.sync_copy(distt_v.at[0], pos_out.at[pl.ds(b0, BW)])
    pltpu.sync_copy(distt_v.at[pl.ds(1, np1 - 1)],
                    negt_out.at[:, pl.ds(b0, BW)])
    pltpu.sync_copy(loss_v, part_out.at[wid])


def _loss_body(p_ref, denom_ref, loss_ref):
    loss_ref[0, 0] = jnp.sum(p_ref[...]) / denom_ref[0]


@jax.jit
def kernel(triple_matrix, entities_emb, relations_emb):
    b, np1, _ = triple_matrix.shape
    neg_n = np1 - 1
    dim = entities_emb.shape[1]

    tri_t = jnp.transpose(triple_matrix, (2, 1, 0))
    h_t = tri_t[0]
    r_t = tri_t[1]
    t_t = tri_t[2]

    ent128 = jnp.pad(entities_emb, ((0, 0), (0, PADDIM - dim)))
    rel128 = jnp.pad(relations_emb, ((0, 0), (0, PADDIM - dim)))

    mesh = plsc.VectorSubcoreMesh(core_axis_name="c", subcore_axis_name="s")
    pos, negt, partials = pl.kernel(
        _sc_distance_body,
        out_type=(
            jax.ShapeDtypeStruct((b,), jnp.float32),
            jax.ShapeDtypeStruct((neg_n, b), jnp.float32),
            jax.ShapeDtypeStruct((NW, LANES), jnp.float32),
        ),
        mesh=mesh,
        compiler_params=pltpu.CompilerParams(
            needs_layout_passes=False, use_tc_tiling_on_sc=True),
        scratch_types=[
            pltpu.VMEM((np1, BW), jnp.int32),
            pltpu.VMEM((np1, BW), jnp.int32),
            pltpu.VMEM((np1, BW), jnp.int32),
            pltpu.VMEM((np1, BW), jnp.float32),
            pltpu.VMEM((LANES,), jnp.float32),
            pltpu.VMEM((CW, PADDIM), jnp.float32),
            pltpu.VMEM((CW, PADDIM), jnp.float32),
            pltpu.VMEM((CW, PADDIM), jnp.float32),
            pltpu.VMEM((CW, PADDIM), jnp.float32),
            pltpu.VMEM((CW, PADDIM), jnp.float32),
            pltpu.VMEM((CW, PADDIM), jnp.float32),
            pltpu.SemaphoreType.DMA,
            pltpu.SemaphoreType.DMA,
            pltpu.SemaphoreType.DMA,
        ],
    )(h_t, r_t, t_t, ent128, rel128)

    denom = jnp.full((1,), float(b * neg_n), jnp.float32)
    loss = pl.pallas_call(
        _loss_body,
        out_shape=jax.ShapeDtypeStruct((1, 1), jnp.float32),
        in_specs=[pl.BlockSpec(memory_space=pltpu.VMEM),
                  pl.BlockSpec(memory_space=pltpu.SMEM)],
        out_specs=pl.BlockSpec(memory_space=pltpu.SMEM),
    )(partials, denom)[0, 0]

    return (loss, pos, negt.T)
